# Initial kernel scaffold; baseline (speedup 1.0000x reference)
import jax
import jax.numpy as jnp
from jax.experimental import pallas as pl


def kernel(x, edge_index, W0, bias, p, Wih, Whh, bih, bhh):
    raise NotImplementedError("write your pallas kernel here")


# speedup vs baseline: 1.0037x; 1.0037x over previous
import functools

import jax
import jax.numpy as jnp
from jax import lax
from jax.experimental import pallas as pl
from jax.experimental.pallas import tpu as pltpu
from jax.experimental.pallas import tpu_sc as plsc

_B, _T, _N, _E = 1, 4, 10000, 320000
_DIN = 128
_DOUT = 128

_NC, _NS = 2, 16
_NW = _NC * _NS

_YROWS = 10240
_RPT = _YROWS // _NW
_TRASH = _RPT
_ACCR = _RPT + 8
_CH = 128
_CPB = 94
_MAXB = _CPB * _CH


def _mesh():
    return plsc.VectorSubcoreMesh(
        core_axis_name="c", subcore_axis_name="s", num_cores=_NC, num_subcores=_NS
    )


def _sc_deg(didb):
    @functools.partial(
        pl.kernel,
        out_type=jax.ShapeDtypeStruct((_YROWS,), jnp.float32),
        mesh=_mesh(),
        compiler_params=pltpu.CompilerParams(needs_layout_passes=False),
        scratch_types=[
            pltpu.VMEM((_MAXB,), jnp.int32),
            pltpu.VMEM((_ACCR,), jnp.float32),
        ],
    )
    def k(didb_hbm, out_hbm, didx_v, acc_v):
        core = lax.axis_index("c")
        sid = lax.axis_index("s")
        w = core * _NS + sid

        pltpu.sync_copy(didb_hbm.at[w], didx_v)

        def fz(i, _):
            acc_v[pl.ds(i * 16, 16)] = jnp.zeros((16,), jnp.float32)
            return 0

        lax.fori_loop(0, _ACCR // 16, fz, 0, unroll=4)

        ones = jnp.full((16,), 1.0, jnp.float32)

        def group(g, _):
            dvec = didx_v[pl.ds(g * 16, 16)]
            plsc.addupdate_scatter(acc_v, [dvec], ones)
            return 0

        lax.fori_loop(0, _MAXB // 16, group, 0)

        pltpu.sync_copy(
            acc_v.at[pl.ds(0, _RPT)], out_hbm.at[pl.ds(w * _RPT, _RPT)]
        )

    return k(didb)


def _tc_prep_body(degp_ref, x_ref, p_ref, dis_ref, xs_ref, s_ref):
    deg1 = degp_ref[...] + 1.0
    dis = lax.rsqrt(deg1)
    dis_ref[...] = dis
    xs_ref[0] = dis * x_ref[0]
    pvec = p_ref[...]
    pn = lax.rsqrt(jnp.sum(pvec * pvec))
    s_ref[0] = jnp.dot(
        x_ref[0], pvec, preferred_element_type=jnp.float32
    ) * pn


def _tc_prep(degp, xpad, p):
    return pl.pallas_call(
        _tc_prep_body,
        grid=(_T,),
        in_specs=[
            pl.BlockSpec((_YROWS, 1), lambda t: (0, 0)),
            pl.BlockSpec((1, _YROWS, _DIN), lambda t: (t, 0, 0)),
            pl.BlockSpec((_DIN, 1), lambda t: (0, 0)),
        ],
        out_specs=[
            pl.BlockSpec((_YROWS, 1), lambda t: (0, 0)),
            pl.BlockSpec((1, _YROWS, _DIN), lambda t: (t, 0, 0)),
            pl.BlockSpec((1, _YROWS, 1), lambda t: (t, 0, 0)),
        ],
        out_shape=[
            jax.ShapeDtypeStruct((_YROWS, 1), jnp.float32),
            jax.ShapeDtypeStruct((_T, _YROWS, _DIN), jnp.float32),
            jax.ShapeDtypeStruct((_T, _YROWS, 1), jnp.float32),
        ],
    )(degp, xpad, p)


def _sc_agg(xsflat, sidb4, didb):
    @functools.partial(
        pl.kernel,
        out_type=jax.ShapeDtypeStruct((_T * _YROWS, _DIN), jnp.float32),
        mesh=_mesh(),
        compiler_params=pltpu.CompilerParams(needs_layout_passes=False),
        scratch_types=[
            pltpu.VMEM((_CPB, _CH), jnp.int32),
            pltpu.VMEM((_MAXB,), jnp.int32),
            pltpu.VMEM((_CH, _DIN), jnp.float32),
            pltpu.VMEM((_ACCR, _DIN), jnp.float32),
            pltpu.SemaphoreType.DMA,
        ],
    )
    def k(xs_hbm, sidb_hbm, didb_hbm, out_hbm, sidx_v, didx_v, rows_v, acc_v, sem):
        core = lax.axis_index("c")
        sid = lax.axis_index("s")
        w = core * _NS + sid

        pltpu.sync_copy(didb_hbm.at[w], didx_v)

        for t in range(_T):
            pltpu.sync_copy(sidb_hbm.at[t * _NW + w], sidx_v)

            def fz(i, _):
                for kk in range(_DIN // 16):
                    acc_v[i, pl.ds(kk * 16, 16)] = jnp.zeros((16,), jnp.float32)
                return 0

            lax.fori_loop(0, _ACCR, fz, 0, unroll=4)

            def chunk(c, _):
                pltpu.async_copy(xs_hbm.at[sidx_v.at[c]], rows_v, sem).wait()

                def group(g, _):
                    dvec = didx_v[pl.ds(c * _CH + g * 16, 16)]
                    evec = lax.iota(jnp.int32, 16) + g * 16
                    for ch in range(_DIN):
                        chv = jnp.full((16,), ch, jnp.int32)
                        vals = plsc.load_gather(rows_v, [evec, chv])
                        plsc.addupdate_scatter(acc_v, [dvec, chv], vals)
                    return 0

                lax.fori_loop(0, _CH // 16, group, 0)
                return 0

            lax.fori_loop(0, _CPB, chunk, 0)

            pltpu.sync_copy(
                acc_v.at[pl.ds(0, _RPT)],
                out_hbm.at[pl.ds(t * _YROWS + w * _RPT, _RPT)],
            )

    return k(xsflat, sidb4, didb)


_NEG = -1e30
_SROWS = _YROWS // _DIN


def _tc_main_body(
    x_ref, y_ref, dis_ref, s2d_ref, w0t_ref, wih_ref, whh_ref, bih_ref,
    bhh_ref, bias_ref, out_ref, h_ref, sc_ref, pool_ref
):
    t = pl.program_id(0)

    @pl.when(t == 0)
    def _():
        h_ref[...] = w0t_ref[...]

    x_t = x_ref[0]

    flat = (
        lax.broadcasted_iota(jnp.int32, (_SROWS, _DIN), 0) * _DIN
        + lax.broadcasted_iota(jnp.int32, (_SROWS, _DIN), 1)
    )
    sc_ref[...] = jnp.where(flat < _N, s2d_ref[0], _NEG)

    def topk_step(i, _):
        scv = sc_ref[...]
        m = jnp.max(scv)
        idx = jnp.min(jnp.where(scv == m, flat, jnp.int32(2**30)))
        row = x_ref[0, pl.ds(idx, 1), :]
        pool_ref[pl.ds(i, 1), :] = row * jnp.tanh(m)
        sc_ref[...] = jnp.where(flat == idx, _NEG, scv)
        return 0

    lax.fori_loop(0, _DOUT, topk_step, 0)

    pooled = pool_ref[...]
    h = h_ref[...]
    gi = lax.dot_general(
        pooled, wih_ref[...], (((1,), (1,)), ((), ())),
        preferred_element_type=jnp.float32,
    ) + bih_ref[...]
    gh = lax.dot_general(
        h, whh_ref[...], (((1,), (1,)), ((), ())),
        preferred_element_type=jnp.float32,
    ) + bhh_ref[...]
    i_r, i_z, i_n = gi[:, :_DIN], gi[:, _DIN:2 * _DIN], gi[:, 2 * _DIN:]
    h_r, h_z, h_n = gh[:, :_DIN], gh[:, _DIN:2 * _DIN], gh[:, 2 * _DIN:]
    r = jax.nn.sigmoid(i_r + h_r)
    z = jax.nn.sigmoid(i_z + h_z)
    n = jnp.tanh(i_n + r * h_n)
    hn = (1.0 - z) * n + z * h
    h_ref[...] = hn

    dis = dis_ref[...]
    zmat = dis * y_ref[0] + (dis * dis) * x_t
    out = lax.dot_general(
        zmat, hn, (((1,), (1,)), ((), ())),
        preferred_element_type=jnp.float32,
    ) + bias_ref[...]
    out_ref[0] = jnp.maximum(out, 0.0)


def _tc_main(xpad, y, dis, s2d, w0t, wih, whh, bih, bhh, bias):
    return pl.pallas_call(
        _tc_main_body,
        grid=(_T,),
        in_specs=[
            pl.BlockSpec((1, _YROWS, _DIN), lambda t: (t, 0, 0)),
            pl.BlockSpec((1, _YROWS, _DIN), lambda t: (t, 0, 0)),
            pl.BlockSpec((_YROWS, 1), lambda t: (0, 0)),
            pl.BlockSpec((1, _SROWS, _DIN), lambda t: (t, 0, 0)),
            pl.BlockSpec((_DOUT, _DIN), lambda t: (0, 0)),
            pl.BlockSpec((3 * _DIN, _DIN), lambda t: (0, 0)),
            pl.BlockSpec((3 * _DIN, _DIN), lambda t: (0, 0)),
            pl.BlockSpec((1, 3 * _DIN), lambda t: (0, 0)),
            pl.BlockSpec((1, 3 * _DIN), lambda t: (0, 0)),
            pl.BlockSpec((1, _DOUT), lambda t: (0, 0)),
        ],
        out_specs=pl.BlockSpec((1, _YROWS, _DOUT), lambda t: (t, 0, 0)),
        out_shape=jax.ShapeDtypeStruct((_T, _YROWS, _DOUT), jnp.float32),
        scratch_shapes=[
            pltpu.VMEM((_DOUT, _DIN), jnp.float32),
            pltpu.VMEM((_SROWS, _DIN), jnp.float32),
            pltpu.VMEM((_DOUT, _DIN), jnp.float32),
        ],
    )(xpad, y, dis, s2d, w0t, wih, whh, bih, bhh, bias)


def kernel(x, edge_index, W0, bias, p, Wih, Whh, bih, bhh):
    src = edge_index[0].astype(jnp.int32)
    dst = edge_index[1].astype(jnp.int32)

    order = jnp.argsort(dst, stable=True)
    key = dst // _RPT
    counts = jnp.bincount(key, length=_NW)
    starts = jnp.concatenate(
        [jnp.zeros((1,), counts.dtype), jnp.cumsum(counts)[:-1]]
    )
    ar = jnp.arange(_MAXB)
    pos = starts[:, None] + ar[None, :]
    valid = ar[None, :] < counts[:, None]
    eid = order[jnp.clip(pos, 0, _E - 1)]
    sb = jnp.where(valid, src[eid], _N)
    db = jnp.where(valid, dst[eid] % _RPT, _TRASH).astype(jnp.int32)
    _GPB = _MAXB // 16
    sb = sb.reshape(_NW, 16, _GPB).transpose(0, 2, 1).reshape(_NW, _MAXB)
    db = db.reshape(_NW, 16, _GPB).transpose(0, 2, 1).reshape(_NW, _MAXB)
    didb = db
    sidb4 = (
        sb[None, :, :] + (jnp.arange(_T, dtype=jnp.int32) * _YROWS)[:, None, None]
    ).astype(jnp.int32).reshape(_T * _NW, _CPB, _CH)

    xpad = jnp.pad(x[0], ((0, 0), (0, _YROWS - _N), (0, 0)))

    degp = _sc_deg(didb)
    dis, xs, scs = _tc_prep(degp.reshape(_YROWS, 1), xpad, p)
    yflat = _sc_agg(xs.reshape(_T * _YROWS, _DIN), sidb4, didb)
    out = _tc_main(
        xpad, yflat.reshape(_T, _YROWS, _DIN), dis,
        scs.reshape(_T, _SROWS, _DIN), W0.T, Wih, Whh,
        bih.reshape(1, -1), bhh.reshape(1, -1), bias.reshape(1, -1),
    )
    return out[None, :, :_N, :]
